# baseline (speedup 1.0000x reference)
import functools
import jax
import jax.numpy as jnp
from jax import lax
from jax.experimental import pallas as pl
from jax.experimental.pallas import tpu as pltpu
from jax.experimental.pallas import tpu_sc as plsc

_BLK = 2048
_R_SC = 2048
_NW = 32
_RW = _R_SC // _NW
_RU = 4


def _tab_kernel(emb_ref, w2_ref, b_ref, aux_ref):
    tab = jax.lax.dot_general(
        emb_ref[...], w2_ref[...], (((1,), (1,)), ((), ())),
        preferred_element_type=jnp.float32)
    aux_ref[0:1, :] = tab[0:1, :] + b_ref[...]
    aux_ref[1:2, :] = tab[1:2, :] - tab[0:1, :]


def _tc_kernel(x_ref, ids_ref, aux_ref, w1_ref, out_ref):
    m = jax.lax.dot_general(
        x_ref[...], w1_ref[...], (((1,), (0,)), ((), ())),
        preferred_element_type=jnp.float32)
    ids = ids_ref[...].astype(jnp.float32)
    out_ref[...] = m + aux_ref[0:1, :] + ids * aux_ref[1:2, :]


def _sc_body(xf_hbm, wt_hbm, aux_hbm, idsf_hbm, out_hbm,
             xv, wv, auxv, idv, ov, h):
    nc = 16
    wid = lax.axis_index("s") * 2 + lax.axis_index("c")
    base = wid * _RW
    gbase = xf_hbm.shape[0] // h - _R_SC + base
    pltpu.sync_copy(xf_hbm.at[pl.ds(gbase * h, _RW * h)], xv)
    pltpu.sync_copy(wt_hbm, wv)
    pltpu.sync_copy(aux_hbm, auxv)
    pltpu.sync_copy(idsf_hbm.at[pl.ds(gbase, _RW)], idv)
    a0 = auxv[pl.ds(0, 16)]
    a1 = auxv[pl.ds(16, 16)]
    zero = jnp.zeros((16,), jnp.float32)

    def grp_body(g, _):
        r0 = g * 16
        idg = idv[pl.ds(r0, 16)]
        for sb in range(16 // _RU):
            r = r0 + sb * _RU

            def kc_body(kc, accs):
                new = list(accs)
                xcs = [xv[pl.ds((r + t) * h + kc * 16, 16)]
                       for t in range(_RU)]
                for j in range(16):
                    wk = wv[pl.ds((kc * 16 + j) * nc, 16)]
                    for t in range(_RU):
                        new[t] = new[t] + xcs[t][j] * wk
                return tuple(new)

            accs = lax.fori_loop(0, h // 16, kc_body, (zero,) * _RU)
            for t in range(_RU):
                ov[pl.ds((r + t) * nc, 16)] = (
                    accs[t] + a0 + idg[sb * _RU + t] * a1)
        return 0

    lax.fori_loop(0, _RW // 16, grp_body, 0)
    pltpu.sync_copy(ov, out_hbm.at[pl.ds(base * nc, _RW * nc)])


def kernel(input, is_predicate_id, emb_table, W, b):
    B, S, H = input.shape
    NC, HD = W.shape
    R = B * S
    r_tc = R - _R_SC
    x = input.reshape(R, H)
    ids = is_predicate_id.reshape(R)
    ids_tc = ids[:r_tc].reshape(r_tc, 1).astype(jnp.int32)
    idsf = ids.astype(jnp.float32)
    w1t = W[:, :H].T
    w2 = W[:, H:]
    b2 = b.reshape(1, NC)

    aux = pl.pallas_call(
        _tab_kernel,
        out_shape=jax.ShapeDtypeStruct((2, NC), jnp.float32),
    )(emb_table, w2, b2)

    tc_out = pl.pallas_call(
        _tc_kernel,
        grid=(r_tc // _BLK,),
        in_specs=[
            pl.BlockSpec((_BLK, H), lambda i: (i, 0)),
            pl.BlockSpec((_BLK, 1), lambda i: (i, 0)),
            pl.BlockSpec((2, NC), lambda i: (0, 0)),
            pl.BlockSpec((H, NC), lambda i: (0, 0)),
        ],
        out_specs=pl.BlockSpec((_BLK, NC), lambda i: (i, 0)),
        out_shape=jax.ShapeDtypeStruct((r_tc, NC), jnp.float32),
        compiler_params=pltpu.CompilerParams(
            dimension_semantics=("arbitrary",)),
    )(x, ids_tc, aux, w1t)

    sc_fn = pl.kernel(
        functools.partial(_sc_body, h=H),
        mesh=plsc.VectorSubcoreMesh(core_axis_name="c", subcore_axis_name="s"),
        out_type=jax.ShapeDtypeStruct((_R_SC * NC,), jnp.float32),
        scratch_types=[
            pltpu.VMEM((_RW * H,), jnp.float32),
            pltpu.VMEM((H * NC,), jnp.float32),
            pltpu.VMEM((2 * NC,), jnp.float32),
            pltpu.VMEM((_RW,), jnp.float32),
            pltpu.VMEM((_RW * NC,), jnp.float32),
        ],
    )
    sc_out = sc_fn(x.reshape(R * H), w1t.reshape(H * NC),
                   aux.reshape(2 * NC), idsf)

    out = jnp.concatenate([tc_out, sc_out.reshape(_R_SC, NC)], axis=0)
    return out.reshape(B, S, NC)


# speedup vs baseline: 1.5234x; 1.5234x over previous
import functools
import jax
import jax.numpy as jnp
from jax import lax
from jax.experimental import pallas as pl
from jax.experimental.pallas import tpu as pltpu
from jax.experimental.pallas import tpu_sc as plsc

_BLK = 1920
_R_SC = 512
_NW = 32
_RW = _R_SC // _NW
_RU = 4


def _tab_kernel(emb_ref, w2_ref, b_ref, aux_ref):
    tab = jax.lax.dot_general(
        emb_ref[...], w2_ref[...], (((1,), (1,)), ((), ())),
        preferred_element_type=jnp.float32)
    aux_ref[0:1, :] = tab[0:1, :] + b_ref[...]
    aux_ref[1:2, :] = tab[1:2, :] - tab[0:1, :]


def _tc_kernel(x_ref, ids_ref, aux_ref, w1_ref, out_ref):
    m = jax.lax.dot_general(
        x_ref[...], w1_ref[...], (((1,), (0,)), ((), ())),
        preferred_element_type=jnp.float32)
    ids = ids_ref[...].astype(jnp.float32)
    out_ref[...] = m + aux_ref[0:1, :] + ids * aux_ref[1:2, :]


def _sc_body(xf_hbm, wt_hbm, aux_hbm, idsf_hbm, out_hbm,
             xv, wv, auxv, idv, ov, h):
    nc = 16
    wid = lax.axis_index("s") * 2 + lax.axis_index("c")
    base = wid * _RW
    gbase = xf_hbm.shape[0] // h - _R_SC + base
    pltpu.sync_copy(xf_hbm.at[pl.ds(gbase * h, _RW * h)], xv)
    pltpu.sync_copy(wt_hbm, wv)
    pltpu.sync_copy(aux_hbm, auxv)
    pltpu.sync_copy(idsf_hbm.at[pl.ds(gbase, _RW)], idv)
    a0 = auxv[pl.ds(0, 16)]
    a1 = auxv[pl.ds(16, 16)]
    zero = jnp.zeros((16,), jnp.float32)

    def grp_body(g, _):
        r0 = g * 16
        idg = idv[pl.ds(r0, 16)]
        for sb in range(16 // _RU):
            r = r0 + sb * _RU

            def kc_body(kc, accs):
                new = list(accs)
                xcs = [xv[pl.ds((r + t) * h + kc * 16, 16)]
                       for t in range(_RU)]
                for j in range(16):
                    wk = wv[pl.ds((kc * 16 + j) * nc, 16)]
                    for t in range(_RU):
                        new[t] = new[t] + xcs[t][j] * wk
                return tuple(new)

            accs = lax.fori_loop(0, h // 16, kc_body, (zero,) * _RU)
            for t in range(_RU):
                ov[pl.ds((r + t) * nc, 16)] = (
                    accs[t] + a0 + idg[sb * _RU + t] * a1)
        return 0

    lax.fori_loop(0, _RW // 16, grp_body, 0)
    pltpu.sync_copy(ov, out_hbm.at[pl.ds(base * nc, _RW * nc)])


def kernel(input, is_predicate_id, emb_table, W, b):
    B, S, H = input.shape
    NC, HD = W.shape
    R = B * S
    r_tc = R - _R_SC
    x = input.reshape(R, H)
    ids = is_predicate_id.reshape(R)
    ids_tc = ids[:r_tc].reshape(r_tc, 1).astype(jnp.int32)
    idsf = ids.astype(jnp.float32)
    w1t = W[:, :H].T
    w2 = W[:, H:]
    b2 = b.reshape(1, NC)

    aux = pl.pallas_call(
        _tab_kernel,
        out_shape=jax.ShapeDtypeStruct((2, NC), jnp.float32),
    )(emb_table, w2, b2)

    tc_out = pl.pallas_call(
        _tc_kernel,
        grid=(r_tc // _BLK,),
        in_specs=[
            pl.BlockSpec((_BLK, H), lambda i: (i, 0)),
            pl.BlockSpec((_BLK, 1), lambda i: (i, 0)),
            pl.BlockSpec((2, NC), lambda i: (0, 0)),
            pl.BlockSpec((H, NC), lambda i: (0, 0)),
        ],
        out_specs=pl.BlockSpec((_BLK, NC), lambda i: (i, 0)),
        out_shape=jax.ShapeDtypeStruct((r_tc, NC), jnp.float32),
        compiler_params=pltpu.CompilerParams(
            dimension_semantics=("arbitrary",)),
    )(x, ids_tc, aux, w1t)

    sc_fn = pl.kernel(
        functools.partial(_sc_body, h=H),
        mesh=plsc.VectorSubcoreMesh(core_axis_name="c", subcore_axis_name="s"),
        out_type=jax.ShapeDtypeStruct((_R_SC * NC,), jnp.float32),
        scratch_types=[
            pltpu.VMEM((_RW * H,), jnp.float32),
            pltpu.VMEM((H * NC,), jnp.float32),
            pltpu.VMEM((2 * NC,), jnp.float32),
            pltpu.VMEM((_RW,), jnp.float32),
            pltpu.VMEM((_RW * NC,), jnp.float32),
        ],
    )
    sc_out = sc_fn(x.reshape(R * H), w1t.reshape(H * NC),
                   aux.reshape(2 * NC), idsf)

    out = jnp.concatenate([tc_out, sc_out.reshape(_R_SC, NC)], axis=0)
    return out.reshape(B, S, NC)


# speedup vs baseline: 5.5978x; 3.6746x over previous
import jax
import jax.numpy as jnp
from jax.experimental import pallas as pl
from jax.experimental.pallas import tpu as pltpu

_BLK = 2048


def _fused_kernel(x_ref, ids_ref, emb_ref, w_ref, b_ref, out_ref):
    h = x_ref.shape[1]
    x = x_ref[...]
    w1 = w_ref[:, :h]
    w2 = w_ref[:, h:]
    tab = jax.lax.dot_general(
        emb_ref[...], w2, (((1,), (1,)), ((), ())),
        preferred_element_type=jnp.float32)
    m = jax.lax.dot_general(
        x, w1, (((1,), (1,)), ((), ())),
        preferred_element_type=jnp.float32)
    ids = ids_ref[...].astype(jnp.float32)
    contrib = tab[0][None, :] + ids * (tab[1] - tab[0])[None, :]
    out_ref[...] = m + contrib + b_ref[...]


def kernel(input, is_predicate_id, emb_table, W, b):
    B, S, H = input.shape
    NC, HD = W.shape
    R = B * S
    x = input.reshape(R, H)
    ids = is_predicate_id.reshape(R, 1).astype(jnp.int32)
    b2 = b.reshape(1, NC)
    grid = (R // _BLK,)
    out = pl.pallas_call(
        _fused_kernel,
        grid=grid,
        in_specs=[
            pl.BlockSpec((_BLK, H), lambda i: (i, 0)),
            pl.BlockSpec((_BLK, 1), lambda i: (i, 0)),
            pl.BlockSpec((2, HD - H), lambda i: (0, 0)),
            pl.BlockSpec((NC, HD), lambda i: (0, 0)),
            pl.BlockSpec((1, NC), lambda i: (0, 0)),
        ],
        out_specs=pl.BlockSpec((_BLK, NC), lambda i: (i, 0)),
        out_shape=jax.ShapeDtypeStruct((R, NC), jnp.float32),
        compiler_params=pltpu.CompilerParams(
            dimension_semantics=("arbitrary",)),
    )(x, ids, emb_table, W, b2)
    return out.reshape(B, S, NC)
